# Initial kernel scaffold; baseline (speedup 1.0000x reference)
import jax
import jax.numpy as jnp
from jax.experimental import pallas as pl


def kernel(feats, edge_index, W1_0, b1_0, W2_0, b2_0, gamma_0, beta_0, W1_1, b1_1, W2_1, b2_1, gamma_1, beta_1):
    raise NotImplementedError("write your pallas kernel here")


# speedup vs baseline: 2.8656x; 2.8656x over previous
import dataclasses
import functools

import jax
import jax.numpy as jnp
from jax import lax
from jax.experimental import pallas as pl
from jax.experimental.pallas import tpu as pltpu
from jax.experimental.pallas import tpu_sc as plsc

_N = 10000
_E = 320000
_D = 128
_H = 128

_NC = 2
_NS = 16
_LANE = 128

_NPAD = 10112
_NB = _NPAD // 128
_NROWS_TILE = _NPAD // _NS
_EROWS = 2560
_EPAD = _EROWS * _LANE
_RPT = _EROWS // (_NC * _NS)
_KB = 8
_FB = _KB * _LANE

_MESH = plsc.VectorSubcoreMesh(core_axis_name="c", subcore_axis_name="s",
                               num_cores=_NC, num_subcores=_NS)

_SC_CP = pltpu.CompilerParams()
if "needs_layout_passes" in pltpu.CompilerParams.__dataclass_fields__:
    _SC_CP = dataclasses.replace(_SC_CP, needs_layout_passes=False)


def _sc_pool_body(h_hbm, src_hbm, dst_hbm, pooled_out, src_v, dst_v, rows_v, acc_sh):
    cid = lax.axis_index("c")
    sid = lax.axis_index("s")
    zero16 = jnp.zeros((16,), jnp.float32)

    @pl.loop(0, _LANE)
    def _(r):
        @pl.loop(0, _D, step=16)
        def _(c):
            rows_v[r, pl.ds(c, 16)] = zero16

    row0 = sid * _NROWS_TILE
    _nfull = _NROWS_TILE // _LANE
    _tail = _NROWS_TILE % _LANE

    @pl.loop(0, _nfull)
    def _(k):
        pltpu.sync_copy(rows_v, acc_sh.at[pl.ds(row0 + k * _LANE, _LANE)])

    if _tail:
        pltpu.sync_copy(rows_v.at[pl.ds(0, _tail)],
                        acc_sh.at[pl.ds(row0 + _nfull * _LANE, _tail)])

    plsc.subcore_barrier()

    base = (cid * _NS + sid) * _RPT

    @pl.loop(0, _RPT // _KB)
    def _(b):
        pltpu.sync_copy(src_hbm.at[pl.ds(base + b * _KB, _KB)], src_v)
        pltpu.sync_copy(dst_hbm.at[pl.ds(base + b * _KB, _KB)], dst_v)

        @pl.loop(0, _KB)
        def _(j):
            pltpu.sync_copy(h_hbm.at[src_v.at[j]], rows_v)
            pltpu.sync_copy(rows_v, acc_sh.at[dst_v.at[j]], add=True)

    plsc.subcore_barrier()

    @pl.loop(0, _nfull)
    def _(k):
        pltpu.sync_copy(acc_sh.at[pl.ds(row0 + k * _LANE, _LANE)],
                        pooled_out.at[cid].at[pl.ds(row0 + k * _LANE, _LANE)])

    if _tail:
        pltpu.sync_copy(acc_sh.at[pl.ds(row0 + _nfull * _LANE, _tail)],
                        pooled_out.at[cid].at[pl.ds(row0 + _nfull * _LANE, _tail)])


_sc_pool = pl.kernel(
    _sc_pool_body,
    out_type=jax.ShapeDtypeStruct((_NC, _NPAD, _D), jnp.float32),
    mesh=_MESH,
    scratch_types=[
        pltpu.VMEM((_KB, _LANE), jnp.int32),
        pltpu.VMEM((_KB, _LANE), jnp.int32),
        pltpu.VMEM((_LANE, _D), jnp.float32),
        pltpu.VMEM_SHARED((_NPAD, _D), jnp.float32),
    ],
)


def _sc_deg_body(dstf_hbm, deg_out, dst_f, hist_v, red_a, red_b, stage_sh):
    cid = lax.axis_index("c")
    sid = lax.axis_index("s")
    zero16 = jnp.zeros((16,), jnp.float32)
    one16 = jnp.ones((16,), jnp.float32)

    @pl.loop(0, _NPAD, step=16)
    def _(c):
        hist_v[pl.ds(c, 16)] = zero16

    ebase = (cid * _NS + sid) * _RPT * _LANE

    @pl.loop(0, _RPT // _KB)
    def _(b):
        pltpu.sync_copy(dstf_hbm.at[pl.ds(ebase + b * _FB, _FB)], dst_f)

        @pl.loop(0, _FB, step=16)
        def _(c):
            plsc.addupdate_scatter(hist_v, [dst_f[pl.ds(c, 16)]], one16)

    pltpu.sync_copy(hist_v, stage_sh.at[pl.ds(sid * _NPAD, _NPAD)])
    plsc.subcore_barrier()

    @pl.loop(0, 5)
    def _(k):
        blk = sid * 5 + k

        @pl.when(blk < _NB)
        def _():
            off = blk * _LANE
            pltpu.sync_copy(stage_sh.at[pl.ds(off, _LANE)], red_a)

            @pl.loop(1, _NS)
            def _(t):
                pltpu.sync_copy(stage_sh.at[pl.ds(t * _NPAD + off, _LANE)], red_b)

                @pl.loop(0, _LANE, step=16)
                def _(c):
                    red_a[pl.ds(c, 16)] = red_a[pl.ds(c, 16)] + red_b[pl.ds(c, 16)]

            pltpu.sync_copy(red_a, deg_out.at[pl.ds(cid * _NPAD + off, _LANE)])


_sc_deg = pl.kernel(
    _sc_deg_body,
    out_type=jax.ShapeDtypeStruct((_NC * _NPAD,), jnp.float32),
    mesh=_MESH,
    compiler_params=_SC_CP,
    scratch_types=[
        pltpu.VMEM((_FB,), jnp.int32),
        pltpu.VMEM((_NPAD,), jnp.float32),
        pltpu.VMEM((_LANE,), jnp.float32),
        pltpu.VMEM((_LANE,), jnp.float32),
        pltpu.VMEM_SHARED((_NS * _NPAD,), jnp.float32),
    ],
)


def _mlp_bn_body(p_ref, deg_ref, w1_ref, b1_ref, w2_ref, b2_ref, g_ref, bt_ref, o_ref):
    p = p_ref[0] + p_ref[1]
    deg = jnp.maximum(deg_ref[0] + deg_ref[1], 1.0)
    pooled = (p.reshape(_NB, 128, _D) / deg[:, :, None]).reshape(_NPAD, _D)
    x = jnp.dot(pooled, w1_ref[...], preferred_element_type=jnp.float32) + b1_ref[...]
    x = jnp.maximum(x, 0.0)
    x = jnp.dot(x, w2_ref[...], preferred_element_type=jnp.float32) + b2_ref[...]
    mask = lax.broadcasted_iota(jnp.int32, (_NPAD, 1), 0) < _N
    xm = jnp.where(mask, x, 0.0)
    mean = jnp.sum(xm, axis=0, keepdims=True) * (1.0 / _N)
    d = jnp.where(mask, x - mean, 0.0)
    var = jnp.sum(d * d, axis=0, keepdims=True) * (1.0 / _N)
    y = (x - mean) * lax.rsqrt(var + 1e-5) * g_ref[...] + bt_ref[...]
    o_ref[...] = jnp.maximum(y, 0.0)


_mlp_bn = pl.pallas_call(
    _mlp_bn_body,
    out_shape=jax.ShapeDtypeStruct((_NPAD, _H), jnp.float32),
)


def kernel(feats, edge_index, W1_0, b1_0, W2_0, b2_0, gamma_0, beta_0,
           W1_1, b1_1, W2_1, b2_1, gamma_1, beta_1):
    dst = edge_index[0]
    src = edge_index[1]
    npad = _EPAD - _E
    src_p = jnp.concatenate([src, jnp.zeros((npad,), src.dtype)]).reshape(_EROWS, _LANE)
    dst_p = jnp.concatenate([dst, jnp.full((npad,), _N, dst.dtype)]).reshape(_EROWS, _LANE)

    deg2 = _sc_deg(dst_p.reshape(-1)).reshape(_NC, _NB, _LANE)
    pooled2 = _sc_pool(feats, src_p, dst_p)
    h1 = _mlp_bn(pooled2, deg2, W1_0, b1_0.reshape(1, -1), W2_0, b2_0.reshape(1, -1),
                 gamma_0.reshape(1, -1), beta_0.reshape(1, -1))
    pooled2b = _sc_pool(h1, src_p, dst_p)
    h2 = _mlp_bn(pooled2b, deg2, W1_1, b1_1.reshape(1, -1), W2_1, b2_1.reshape(1, -1),
                 gamma_1.reshape(1, -1), beta_1.reshape(1, -1))
    return h2[:_N]


# speedup vs baseline: 3.1198x; 1.0887x over previous
import dataclasses
import functools

import jax
import jax.numpy as jnp
from jax import lax
from jax.experimental import pallas as pl
from jax.experimental.pallas import tpu as pltpu
from jax.experimental.pallas import tpu_sc as plsc

_N = 10000
_E = 320000
_D = 128
_H = 128

_NC = 2
_NS = 16
_LANE = 128

_NPAD = 10112
_NB = _NPAD // 128
_NROWS_TILE = _NPAD // _NS
_EROWS = 2560
_EPAD = _EROWS * _LANE
_RPT = _EROWS // (_NC * _NS)
_KB = 8
_FB = _KB * _LANE

_MESH = plsc.VectorSubcoreMesh(core_axis_name="c", subcore_axis_name="s",
                               num_cores=_NC, num_subcores=_NS)

_SC_CP = pltpu.CompilerParams()
if "needs_layout_passes" in pltpu.CompilerParams.__dataclass_fields__:
    _SC_CP = dataclasses.replace(_SC_CP, needs_layout_passes=False)


def _sc_pool_body(h_hbm, src_hbm, dst_hbm, pooled_out, src_v, dst_v,
                  rows_a, rows_b, acc_sh, gsa, gsb, ssa, ssb):
    cid = lax.axis_index("c")
    sid = lax.axis_index("s")
    zero16 = jnp.zeros((16,), jnp.float32)

    @pl.loop(0, _LANE)
    def _(r):
        @pl.loop(0, _D, step=16)
        def _(c):
            rows_a[r, pl.ds(c, 16)] = zero16

    row0 = sid * _NROWS_TILE
    _nfull = _NROWS_TILE // _LANE
    _tail = _NROWS_TILE % _LANE

    @pl.loop(0, _nfull)
    def _(k):
        pltpu.sync_copy(rows_a, acc_sh.at[pl.ds(row0 + k * _LANE, _LANE)])

    if _tail:
        pltpu.sync_copy(rows_a.at[pl.ds(0, _tail)],
                        acc_sh.at[pl.ds(row0 + _nfull * _LANE, _tail)])

    plsc.subcore_barrier()

    base = (cid * _NS + sid) * _RPT
    bufs = (rows_a, rows_b)
    gsems = (gsa, gsb)
    ssems = (ssa, ssb)

    @pl.loop(0, _RPT // _KB)
    def _(b):
        pltpu.sync_copy(src_hbm.at[pl.ds(base + b * _KB, _KB)], src_v)
        pltpu.sync_copy(dst_hbm.at[pl.ds(base + b * _KB, _KB)], dst_v)
        pltpu.async_copy(h_hbm.at[src_v.at[0]], bufs[0], gsems[0])
        for j in range(_KB):
            p = j & 1
            q = p ^ 1
            if j + 1 < _KB:
                if j >= 1:
                    pltpu.make_async_copy(bufs[q], acc_sh.at[dst_v.at[j - 1]],
                                          ssems[q]).wait()
                pltpu.async_copy(h_hbm.at[src_v.at[j + 1]], bufs[q], gsems[q])
            pltpu.make_async_copy(h_hbm.at[src_v.at[j]], bufs[p], gsems[p]).wait()
            pltpu.async_copy(bufs[p], acc_sh.at[dst_v.at[j]], ssems[p], add=True)
        for j in (_KB - 2, _KB - 1):
            p = j & 1
            pltpu.make_async_copy(bufs[p], acc_sh.at[dst_v.at[j]], ssems[p]).wait()

    plsc.subcore_barrier()

    @pl.loop(0, _nfull)
    def _(k):
        pltpu.sync_copy(acc_sh.at[pl.ds(row0 + k * _LANE, _LANE)],
                        pooled_out.at[cid].at[pl.ds(row0 + k * _LANE, _LANE)])

    if _tail:
        pltpu.sync_copy(acc_sh.at[pl.ds(row0 + _nfull * _LANE, _tail)],
                        pooled_out.at[cid].at[pl.ds(row0 + _nfull * _LANE, _tail)])


_sc_pool = pl.kernel(
    _sc_pool_body,
    out_type=jax.ShapeDtypeStruct((_NC, _NPAD, _D), jnp.float32),
    mesh=_MESH,
    scratch_types=[
        pltpu.VMEM((_KB, _LANE), jnp.int32),
        pltpu.VMEM((_KB, _LANE), jnp.int32),
        pltpu.VMEM((_LANE, _D), jnp.float32),
        pltpu.VMEM((_LANE, _D), jnp.float32),
        pltpu.VMEM_SHARED((_NPAD, _D), jnp.float32),
        pltpu.SemaphoreType.DMA,
        pltpu.SemaphoreType.DMA,
        pltpu.SemaphoreType.DMA,
        pltpu.SemaphoreType.DMA,
    ],
)


def _sc_deg_body(dstf_hbm, deg_out, dst_f, hist_v, red_a, red_b, stage_sh):
    cid = lax.axis_index("c")
    sid = lax.axis_index("s")
    zero16 = jnp.zeros((16,), jnp.float32)
    one16 = jnp.ones((16,), jnp.float32)

    @pl.loop(0, _NPAD, step=16)
    def _(c):
        hist_v[pl.ds(c, 16)] = zero16

    ebase = (cid * _NS + sid) * _RPT * _LANE

    @pl.loop(0, _RPT // _KB)
    def _(b):
        pltpu.sync_copy(dstf_hbm.at[pl.ds(ebase + b * _FB, _FB)], dst_f)

        @pl.loop(0, _FB, step=16)
        def _(c):
            plsc.addupdate_scatter(hist_v, [dst_f[pl.ds(c, 16)]], one16)

    pltpu.sync_copy(hist_v, stage_sh.at[pl.ds(sid * _NPAD, _NPAD)])
    plsc.subcore_barrier()

    @pl.loop(0, 5)
    def _(k):
        blk = sid * 5 + k

        @pl.when(blk < _NB)
        def _():
            off = blk * _LANE
            pltpu.sync_copy(stage_sh.at[pl.ds(off, _LANE)], red_a)

            @pl.loop(1, _NS)
            def _(t):
                pltpu.sync_copy(stage_sh.at[pl.ds(t * _NPAD + off, _LANE)], red_b)

                @pl.loop(0, _LANE, step=16)
                def _(c):
                    red_a[pl.ds(c, 16)] = red_a[pl.ds(c, 16)] + red_b[pl.ds(c, 16)]

            pltpu.sync_copy(red_a, deg_out.at[pl.ds(cid * _NPAD + off, _LANE)])


_sc_deg = pl.kernel(
    _sc_deg_body,
    out_type=jax.ShapeDtypeStruct((_NC * _NPAD,), jnp.float32),
    mesh=_MESH,
    compiler_params=_SC_CP,
    scratch_types=[
        pltpu.VMEM((_FB,), jnp.int32),
        pltpu.VMEM((_NPAD,), jnp.float32),
        pltpu.VMEM((_LANE,), jnp.float32),
        pltpu.VMEM((_LANE,), jnp.float32),
        pltpu.VMEM_SHARED((_NS * _NPAD,), jnp.float32),
    ],
)


def _mlp_bn_body(p_ref, deg_ref, w1_ref, b1_ref, w2_ref, b2_ref, g_ref, bt_ref, o_ref):
    p = p_ref[0] + p_ref[1]
    deg = jnp.maximum(deg_ref[0] + deg_ref[1], 1.0)
    pooled = (p.reshape(_NB, 128, _D) / deg[:, :, None]).reshape(_NPAD, _D)
    x = jnp.dot(pooled, w1_ref[...], preferred_element_type=jnp.float32) + b1_ref[...]
    x = jnp.maximum(x, 0.0)
    x = jnp.dot(x, w2_ref[...], preferred_element_type=jnp.float32) + b2_ref[...]
    mask = lax.broadcasted_iota(jnp.int32, (_NPAD, 1), 0) < _N
    xm = jnp.where(mask, x, 0.0)
    mean = jnp.sum(xm, axis=0, keepdims=True) * (1.0 / _N)
    d = jnp.where(mask, x - mean, 0.0)
    var = jnp.sum(d * d, axis=0, keepdims=True) * (1.0 / _N)
    y = (x - mean) * lax.rsqrt(var + 1e-5) * g_ref[...] + bt_ref[...]
    o_ref[...] = jnp.maximum(y, 0.0)


_mlp_bn = pl.pallas_call(
    _mlp_bn_body,
    out_shape=jax.ShapeDtypeStruct((_NPAD, _H), jnp.float32),
)


def kernel(feats, edge_index, W1_0, b1_0, W2_0, b2_0, gamma_0, beta_0,
           W1_1, b1_1, W2_1, b2_1, gamma_1, beta_1):
    dst = edge_index[0]
    src = edge_index[1]
    npad = _EPAD - _E
    src_p = jnp.concatenate([src, jnp.zeros((npad,), src.dtype)]).reshape(_EROWS, _LANE)
    dst_p = jnp.concatenate([dst, jnp.full((npad,), _N, dst.dtype)]).reshape(_EROWS, _LANE)

    deg2 = _sc_deg(dst_p.reshape(-1)).reshape(_NC, _NB, _LANE)
    pooled2 = _sc_pool(feats, src_p, dst_p)
    h1 = _mlp_bn(pooled2, deg2, W1_0, b1_0.reshape(1, -1), W2_0, b2_0.reshape(1, -1),
                 gamma_0.reshape(1, -1), beta_0.reshape(1, -1))
    pooled2b = _sc_pool(h1, src_p, dst_p)
    h2 = _mlp_bn(pooled2b, deg2, W1_1, b1_1.reshape(1, -1), W2_1, b2_1.reshape(1, -1),
                 gamma_1.reshape(1, -1), beta_1.reshape(1, -1))
    return h2[:_N]


# speedup vs baseline: 3.3313x; 1.0678x over previous
import dataclasses
import functools

import jax
import jax.numpy as jnp
from jax import lax
from jax.experimental import pallas as pl
from jax.experimental.pallas import tpu as pltpu
from jax.experimental.pallas import tpu_sc as plsc

_N = 10000
_E = 320000
_D = 128
_H = 128

_NC = 2
_NS = 16
_LANE = 128

_NPAD = 10112
_NB = _NPAD // 128
_NROWS_TILE = _NPAD // _NS
_EROWS = 2560
_EPAD = _EROWS * _LANE
_RPT = _EROWS // (_NC * _NS)
_KB = 8
_FB = _KB * _LANE

_MESH = plsc.VectorSubcoreMesh(core_axis_name="c", subcore_axis_name="s",
                               num_cores=_NC, num_subcores=_NS)

_SC_CP = pltpu.CompilerParams()
if "needs_layout_passes" in pltpu.CompilerParams.__dataclass_fields__:
    _SC_CP = dataclasses.replace(_SC_CP, needs_layout_passes=False)


def _sc_pool_body(h_hbm, src_hbm, dst_hbm, pooled_out, src_v, dst_v,
                  rows_a, rows_b, acc_sh, gsa, gsb, ssa, ssb):
    cid = lax.axis_index("c")
    sid = lax.axis_index("s")
    zero16 = jnp.zeros((16,), jnp.float32)

    @pl.loop(0, _LANE)
    def _(r):
        @pl.loop(0, _D, step=16)
        def _(c):
            rows_a[r, pl.ds(c, 16)] = zero16

    row0 = sid * _NROWS_TILE
    _nfull = _NROWS_TILE // _LANE
    _tail = _NROWS_TILE % _LANE

    @pl.loop(0, _nfull)
    def _(k):
        pltpu.sync_copy(rows_a, acc_sh.at[pl.ds(row0 + k * _LANE, _LANE)])

    if _tail:
        pltpu.sync_copy(rows_a.at[pl.ds(0, _tail)],
                        acc_sh.at[pl.ds(row0 + _nfull * _LANE, _tail)])

    plsc.subcore_barrier()

    base = (cid * _NS + sid) * _RPT
    bufs = (rows_a, rows_b)
    gsems = (gsa, gsb)
    ssems = (ssa, ssb)

    @pl.loop(0, _RPT // _KB)
    def _(b):
        pltpu.sync_copy(src_hbm.at[pl.ds(base + b * _KB, _KB)], src_v)
        pltpu.sync_copy(dst_hbm.at[pl.ds(base + b * _KB, _KB)], dst_v)
        pltpu.async_copy(h_hbm.at[src_v.at[0]], bufs[0], gsems[0])
        for j in range(_KB):
            p = j & 1
            q = p ^ 1
            if j + 1 < _KB:
                if j >= 1:
                    pltpu.make_async_copy(bufs[q], acc_sh.at[dst_v.at[j - 1]],
                                          ssems[q]).wait()
                pltpu.async_copy(h_hbm.at[src_v.at[j + 1]], bufs[q], gsems[q])
            pltpu.make_async_copy(h_hbm.at[src_v.at[j]], bufs[p], gsems[p]).wait()
            pltpu.async_copy(bufs[p], acc_sh.at[dst_v.at[j]], ssems[p], add=True)
        for j in (_KB - 2, _KB - 1):
            p = j & 1
            pltpu.make_async_copy(bufs[p], acc_sh.at[dst_v.at[j]], ssems[p]).wait()

    plsc.subcore_barrier()

    @pl.loop(0, _nfull)
    def _(k):
        pltpu.sync_copy(acc_sh.at[pl.ds(row0 + k * _LANE, _LANE)],
                        pooled_out.at[cid].at[pl.ds(row0 + k * _LANE, _LANE)])

    if _tail:
        pltpu.sync_copy(acc_sh.at[pl.ds(row0 + _nfull * _LANE, _tail)],
                        pooled_out.at[cid].at[pl.ds(row0 + _nfull * _LANE, _tail)])


_sc_pool = pl.kernel(
    _sc_pool_body,
    out_type=jax.ShapeDtypeStruct((_NC, _NPAD, _D), jnp.float32),
    mesh=_MESH,
    scratch_types=[
        pltpu.VMEM((_KB, _LANE), jnp.int32),
        pltpu.VMEM((_KB, _LANE), jnp.int32),
        pltpu.VMEM((_LANE, _D), jnp.float32),
        pltpu.VMEM((_LANE, _D), jnp.float32),
        pltpu.VMEM_SHARED((_NPAD, _D), jnp.float32),
        pltpu.SemaphoreType.DMA,
        pltpu.SemaphoreType.DMA,
        pltpu.SemaphoreType.DMA,
        pltpu.SemaphoreType.DMA,
    ],
)


def _sc_deg_body(dstf_hbm, deg_out, dst_f, hist_v, red_a, red_b, stage_sh):
    cid = lax.axis_index("c")
    sid = lax.axis_index("s")
    zero16 = jnp.zeros((16,), jnp.float32)
    one16 = jnp.ones((16,), jnp.float32)

    @pl.loop(0, _NPAD, step=16)
    def _(c):
        hist_v[pl.ds(c, 16)] = zero16

    ebase = (cid * _NS + sid) * _RPT * _LANE

    @pl.loop(0, _RPT // _KB)
    def _(b):
        pltpu.sync_copy(dstf_hbm.at[pl.ds(ebase + b * _FB, _FB)], dst_f)

        @pl.loop(0, _FB, step=16)
        def _(c):
            plsc.addupdate_scatter(hist_v, [dst_f[pl.ds(c, 16)]], one16)

    pltpu.sync_copy(hist_v, stage_sh.at[pl.ds(sid * _NPAD, _NPAD)])
    plsc.subcore_barrier()

    @pl.loop(0, 5)
    def _(k):
        blk = sid * 5 + k

        @pl.when(blk < _NB)
        def _():
            off = blk * _LANE
            pltpu.sync_copy(stage_sh.at[pl.ds(off, _LANE)], red_a)

            @pl.loop(1, _NS)
            def _(t):
                pltpu.sync_copy(stage_sh.at[pl.ds(t * _NPAD + off, _LANE)], red_b)

                @pl.loop(0, _LANE, step=16)
                def _(c):
                    red_a[pl.ds(c, 16)] = red_a[pl.ds(c, 16)] + red_b[pl.ds(c, 16)]

            pltpu.sync_copy(red_a, deg_out.at[pl.ds(cid * _NPAD + off, _LANE)])


_sc_deg = pl.kernel(
    _sc_deg_body,
    out_type=jax.ShapeDtypeStruct((_NC * _NPAD,), jnp.float32),
    mesh=_MESH,
    compiler_params=_SC_CP,
    scratch_types=[
        pltpu.VMEM((_FB,), jnp.int32),
        pltpu.VMEM((_NPAD,), jnp.float32),
        pltpu.VMEM((_LANE,), jnp.float32),
        pltpu.VMEM((_LANE,), jnp.float32),
        pltpu.VMEM_SHARED((_NS * _NPAD,), jnp.float32),
    ],
)


def _mlp_bn_body(p_ref, deg_ref, w1_ref, b1_ref, w2_ref, b2_ref, g_ref, bt_ref, o_ref):
    p = p_ref[0] + p_ref[1]
    deg = jnp.maximum(deg_ref[0] + deg_ref[1], 1.0)
    pooled = (p.reshape(_NB, 128, _D) / deg[:, :, None]).reshape(_NPAD, _D)
    x = jnp.dot(pooled, w1_ref[...], preferred_element_type=jnp.float32) + b1_ref[...]
    x = jnp.maximum(x, 0.0)
    x = jnp.dot(x, w2_ref[...], preferred_element_type=jnp.float32) + b2_ref[...]
    mask = lax.broadcasted_iota(jnp.int32, (_NPAD, 1), 0) < _N
    xm = jnp.where(mask, x, 0.0)
    mean = jnp.sum(xm, axis=0, keepdims=True) * (1.0 / _N)
    d = jnp.where(mask, x - mean, 0.0)
    var = jnp.sum(d * d, axis=0, keepdims=True) * (1.0 / _N)
    y = (x - mean) * lax.rsqrt(var + 1e-5) * g_ref[...] + bt_ref[...]
    o_ref[...] = jnp.maximum(y, 0.0)


_mlp_bn = pl.pallas_call(
    _mlp_bn_body,
    out_shape=jax.ShapeDtypeStruct((_NPAD, _H), jnp.float32),
)


def kernel(feats, edge_index, W1_0, b1_0, W2_0, b2_0, gamma_0, beta_0,
           W1_1, b1_1, W2_1, b2_1, gamma_1, beta_1):
    dst = edge_index[0]
    src = edge_index[1]
    npad = _EPAD - _E
    pad_dst = _N + (jnp.arange(npad, dtype=dst.dtype) % (_NPAD - _N))
    src_p = jnp.concatenate([src, jnp.zeros((npad,), src.dtype)]).reshape(_EROWS, _LANE)
    dst_p = jnp.concatenate([dst, pad_dst]).reshape(_EROWS, _LANE)

    deg2 = _sc_deg(dst_p.reshape(-1)).reshape(_NC, _NB, _LANE)
    pooled2 = _sc_pool(feats, src_p, dst_p)
    h1 = _mlp_bn(pooled2, deg2, W1_0, b1_0.reshape(1, -1), W2_0, b2_0.reshape(1, -1),
                 gamma_0.reshape(1, -1), beta_0.reshape(1, -1))
    pooled2b = _sc_pool(h1, src_p, dst_p)
    h2 = _mlp_bn(pooled2b, deg2, W1_1, b1_1.reshape(1, -1), W2_1, b2_1.reshape(1, -1),
                 gamma_1.reshape(1, -1), beta_1.reshape(1, -1))
    return h2[:_N]


# speedup vs baseline: 3.8699x; 1.1617x over previous
import dataclasses
import functools

import jax
import jax.numpy as jnp
from jax import lax
from jax.experimental import pallas as pl
from jax.experimental.pallas import tpu as pltpu
from jax.experimental.pallas import tpu_sc as plsc

_N = 10000
_E = 320000
_D = 128
_H = 128

_NC = 2
_NS = 16
_LANE = 128

_NPAD = 10112
_NB = _NPAD // 128
_NROWS_TILE = _NPAD // _NS
_EROWS = 2560
_EPAD = _EROWS * _LANE
_RPT = _EROWS // (_NC * _NS)
_KB = 8
_RPT0 = 136
_RPT1 = 2 * _RPT - _RPT0
_FB = _KB * _LANE

_MESH = plsc.VectorSubcoreMesh(core_axis_name="c", subcore_axis_name="s",
                               num_cores=_NC, num_subcores=_NS)

_SC_CP = pltpu.CompilerParams()
if "needs_layout_passes" in pltpu.CompilerParams.__dataclass_fields__:
    _SC_CP = dataclasses.replace(_SC_CP, needs_layout_passes=False)


def _sc_pool_body(h_hbm, src_hbm, dst_hbm, pooled_out, src_v, dst_v,
                  rows_a, rows_b, acc_sh, gsa, gsb, ssa, ssb):
    cid = lax.axis_index("c")
    sid = lax.axis_index("s")
    zero16 = jnp.zeros((16,), jnp.float32)

    @pl.loop(0, _LANE)
    def _(r):
        @pl.loop(0, _D, step=16)
        def _(c):
            rows_a[r, pl.ds(c, 16)] = zero16

    row0 = sid * _NROWS_TILE
    _nfull = _NROWS_TILE // _LANE
    _tail = _NROWS_TILE % _LANE

    @pl.loop(0, _nfull)
    def _(k):
        pltpu.sync_copy(rows_a, acc_sh.at[pl.ds(row0 + k * _LANE, _LANE)])

    if _tail:
        pltpu.sync_copy(rows_a.at[pl.ds(0, _tail)],
                        acc_sh.at[pl.ds(row0 + _nfull * _LANE, _tail)])

    plsc.subcore_barrier()

    base = jnp.where(cid == 0, sid * _RPT0, _NS * _RPT0 + sid * _RPT1)
    nblk = jnp.where(cid == 0, _RPT0 // _KB, _RPT1 // _KB)
    bufs = (rows_a, rows_b)
    gsems = (gsa, gsb)
    ssems = (ssa, ssb)

    @pl.loop(0, max(_RPT0, _RPT1) // _KB)
    def _(b):
        @pl.when(b < nblk)
        def _():
            pltpu.sync_copy(src_hbm.at[pl.ds(base + b * _KB, _KB)], src_v)
            pltpu.sync_copy(dst_hbm.at[pl.ds(base + b * _KB, _KB)], dst_v)
            pltpu.async_copy(h_hbm.at[src_v.at[0]], bufs[0], gsems[0])
            for j in range(_KB):
                p = j & 1
                q = p ^ 1
                if j + 1 < _KB:
                    if j >= 1:
                        pltpu.make_async_copy(bufs[q], acc_sh.at[dst_v.at[j - 1]],
                                              ssems[q]).wait()
                    pltpu.async_copy(h_hbm.at[src_v.at[j + 1]], bufs[q], gsems[q])
                pltpu.make_async_copy(h_hbm.at[src_v.at[j]], bufs[p], gsems[p]).wait()
                pltpu.async_copy(bufs[p], acc_sh.at[dst_v.at[j]], ssems[p], add=True)
            for j in (_KB - 2, _KB - 1):
                p = j & 1
                pltpu.make_async_copy(bufs[p], acc_sh.at[dst_v.at[j]], ssems[p]).wait()

    plsc.subcore_barrier()

    @pl.loop(0, _nfull)
    def _(k):
        pltpu.sync_copy(acc_sh.at[pl.ds(row0 + k * _LANE, _LANE)],
                        pooled_out.at[cid].at[pl.ds(row0 + k * _LANE, _LANE)])

    if _tail:
        pltpu.sync_copy(acc_sh.at[pl.ds(row0 + _nfull * _LANE, _tail)],
                        pooled_out.at[cid].at[pl.ds(row0 + _nfull * _LANE, _tail)])


_sc_pool = pl.kernel(
    _sc_pool_body,
    out_type=jax.ShapeDtypeStruct((_NC, _NPAD, _D), jnp.float32),
    mesh=_MESH,
    scratch_types=[
        pltpu.VMEM((_KB, _LANE), jnp.int32),
        pltpu.VMEM((_KB, _LANE), jnp.int32),
        pltpu.VMEM((_LANE, _D), jnp.float32),
        pltpu.VMEM((_LANE, _D), jnp.float32),
        pltpu.VMEM_SHARED((_NPAD, _D), jnp.float32),
        pltpu.SemaphoreType.DMA,
        pltpu.SemaphoreType.DMA,
        pltpu.SemaphoreType.DMA,
        pltpu.SemaphoreType.DMA,
    ],
)


def _sc_deg_body(dstf_hbm, deg_out, dst_f, hist_v, red_a, red_b, stage_sh):
    cid = lax.axis_index("c")
    sid = lax.axis_index("s")
    zero16 = jnp.zeros((16,), jnp.float32)
    one16 = jnp.ones((16,), jnp.float32)

    @pl.loop(0, _NPAD, step=16)
    def _(c):
        hist_v[pl.ds(c, 16)] = zero16

    ebase = (cid * _NS + sid) * _RPT * _LANE

    @pl.loop(0, _RPT // _KB)
    def _(b):
        pltpu.sync_copy(dstf_hbm.at[pl.ds(ebase + b * _FB, _FB)], dst_f)

        @pl.loop(0, _FB, step=16)
        def _(c):
            plsc.addupdate_scatter(hist_v, [dst_f[pl.ds(c, 16)]], one16)

    pltpu.sync_copy(hist_v, stage_sh.at[pl.ds(sid * _NPAD, _NPAD)])
    plsc.subcore_barrier()

    @pl.loop(0, 5)
    def _(k):
        blk = sid * 5 + k

        @pl.when(blk < _NB)
        def _():
            off = blk * _LANE
            pltpu.sync_copy(stage_sh.at[pl.ds(off, _LANE)], red_a)

            @pl.loop(1, _NS)
            def _(t):
                pltpu.sync_copy(stage_sh.at[pl.ds(t * _NPAD + off, _LANE)], red_b)

                @pl.loop(0, _LANE, step=16)
                def _(c):
                    red_a[pl.ds(c, 16)] = red_a[pl.ds(c, 16)] + red_b[pl.ds(c, 16)]

            pltpu.sync_copy(red_a, deg_out.at[pl.ds(cid * _NPAD + off, _LANE)])


_sc_deg = pl.kernel(
    _sc_deg_body,
    out_type=jax.ShapeDtypeStruct((_NC * _NPAD,), jnp.float32),
    mesh=_MESH,
    compiler_params=_SC_CP,
    scratch_types=[
        pltpu.VMEM((_FB,), jnp.int32),
        pltpu.VMEM((_NPAD,), jnp.float32),
        pltpu.VMEM((_LANE,), jnp.float32),
        pltpu.VMEM((_LANE,), jnp.float32),
        pltpu.VMEM_SHARED((_NS * _NPAD,), jnp.float32),
    ],
)


def _mlp_bn_body(p_ref, deg_ref, w1_ref, b1_ref, w2_ref, b2_ref, g_ref, bt_ref, o_ref):
    p = p_ref[0] + p_ref[1]
    deg = jnp.maximum(deg_ref[0] + deg_ref[1], 1.0)
    pooled = (p.reshape(_NB, 128, _D) / deg[:, :, None]).reshape(_NPAD, _D)
    x = jnp.dot(pooled, w1_ref[...], preferred_element_type=jnp.float32) + b1_ref[...]
    x = jnp.maximum(x, 0.0)
    x = jnp.dot(x, w2_ref[...], preferred_element_type=jnp.float32) + b2_ref[...]
    mask = lax.broadcasted_iota(jnp.int32, (_NPAD, 1), 0) < _N
    xm = jnp.where(mask, x, 0.0)
    mean = jnp.sum(xm, axis=0, keepdims=True) * (1.0 / _N)
    d = jnp.where(mask, x - mean, 0.0)
    var = jnp.sum(d * d, axis=0, keepdims=True) * (1.0 / _N)
    y = (x - mean) * lax.rsqrt(var + 1e-5) * g_ref[...] + bt_ref[...]
    o_ref[...] = jnp.maximum(y, 0.0)


_mlp_bn = pl.pallas_call(
    _mlp_bn_body,
    out_shape=jax.ShapeDtypeStruct((_NPAD, _H), jnp.float32),
)


def kernel(feats, edge_index, W1_0, b1_0, W2_0, b2_0, gamma_0, beta_0,
           W1_1, b1_1, W2_1, b2_1, gamma_1, beta_1):
    dst = edge_index[0]
    src = edge_index[1]
    npad = _EPAD - _E
    pad_dst = _N + (jnp.arange(npad, dtype=dst.dtype) % (_NPAD - _N))
    src_p = jnp.concatenate([src, jnp.zeros((npad,), src.dtype)]).reshape(_EROWS, _LANE)
    dst_p = jnp.concatenate([dst, pad_dst]).reshape(_EROWS, _LANE)

    deg2 = _sc_deg(dst_p.reshape(-1)).reshape(_NC, _NB, _LANE)
    pooled2 = _sc_pool(feats, src_p, dst_p)
    h1 = _mlp_bn(pooled2, deg2, W1_0, b1_0.reshape(1, -1), W2_0, b2_0.reshape(1, -1),
                 gamma_0.reshape(1, -1), beta_0.reshape(1, -1))
    pooled2b = _sc_pool(h1, src_p, dst_p)
    h2 = _mlp_bn(pooled2b, deg2, W1_1, b1_1.reshape(1, -1), W2_1, b2_1.reshape(1, -1),
                 gamma_1.reshape(1, -1), beta_1.reshape(1, -1))
    return h2[:_N]


# speedup vs baseline: 10.4102x; 2.6901x over previous
import dataclasses
import functools

import jax
import jax.numpy as jnp
from jax import lax
from jax.experimental import pallas as pl
from jax.experimental.pallas import tpu as pltpu
from jax.experimental.pallas import tpu_sc as plsc

_N = 10000
_E = 320000
_D = 128
_H = 128

_NC = 2
_NS = 16
_LANE = 128

_NPAD = 10112
_NB = _NPAD // 128
_NROWS_TILE = _NPAD // _NS
_EROWS = 2560
_EPAD = _EROWS * _LANE
_RPT = _EROWS // (_NC * _NS)
_KB = 8
_RPT0 = _RPT
_RPT1 = 2 * _RPT - _RPT0
_FB = _KB * _LANE

_MESH = plsc.VectorSubcoreMesh(core_axis_name="c", subcore_axis_name="s",
                               num_cores=_NC, num_subcores=_NS)

_SC_CP = pltpu.CompilerParams()
if "needs_layout_passes" in pltpu.CompilerParams.__dataclass_fields__:
    _SC_CP = dataclasses.replace(_SC_CP, needs_layout_passes=False)


def _sc_pool_body(h_hbm, src_hbm, dst_hbm, pooled_out, src_v, dst_v,
                  rows_a, rows_b, acc_sh, gsa, gsb, ssa, ssb):
    cid = lax.axis_index("c")
    sid = lax.axis_index("s")
    zero16 = jnp.zeros((16,), jnp.float32)

    @pl.loop(0, _LANE)
    def _(r):
        @pl.loop(0, _D, step=16)
        def _(c):
            rows_a[r, pl.ds(c, 16)] = zero16

    row0 = sid * _NROWS_TILE
    _nfull = _NROWS_TILE // _LANE
    _tail = _NROWS_TILE % _LANE

    @pl.loop(0, _nfull)
    def _(k):
        pltpu.sync_copy(rows_a, acc_sh.at[pl.ds(row0 + k * _LANE, _LANE)])

    if _tail:
        pltpu.sync_copy(rows_a.at[pl.ds(0, _tail)],
                        acc_sh.at[pl.ds(row0 + _nfull * _LANE, _tail)])

    plsc.subcore_barrier()

    base = jnp.where(cid == 0, sid * _RPT0, _NS * _RPT0 + sid * _RPT1)
    nblk = jnp.where(cid == 0, _RPT0 // _KB, _RPT1 // _KB)
    bufs = (rows_a, rows_b)
    gsems = (gsa, gsb)
    ssems = (ssa, ssb)

    @pl.loop(0, max(_RPT0, _RPT1) // _KB)
    def _(b):
        @pl.when(b < nblk)
        def _():
            pltpu.sync_copy(src_hbm.at[pl.ds(base + b * _KB, _KB)], src_v)
            pltpu.sync_copy(dst_hbm.at[pl.ds(base + b * _KB, _KB)], dst_v)
            pltpu.async_copy(h_hbm.at[src_v.at[0]], bufs[0], gsems[0])
            for j in range(_KB):
                p = j & 1
                q = p ^ 1
                if j + 1 < _KB:
                    if j >= 1:
                        pltpu.make_async_copy(bufs[q], acc_sh.at[dst_v.at[j - 1]],
                                              ssems[q]).wait()
                    pltpu.async_copy(h_hbm.at[src_v.at[j + 1]], bufs[q], gsems[q])
                pltpu.make_async_copy(h_hbm.at[src_v.at[j]], bufs[p], gsems[p]).wait()
                pltpu.async_copy(bufs[p], acc_sh.at[dst_v.at[j]], ssems[p], add=True)
            for j in (_KB - 2, _KB - 1):
                p = j & 1
                pltpu.make_async_copy(bufs[p], acc_sh.at[dst_v.at[j]], ssems[p]).wait()

    plsc.subcore_barrier()

    @pl.loop(0, _nfull)
    def _(k):
        pltpu.sync_copy(acc_sh.at[pl.ds(row0 + k * _LANE, _LANE)],
                        pooled_out.at[cid].at[pl.ds(row0 + k * _LANE, _LANE)])

    if _tail:
        pltpu.sync_copy(acc_sh.at[pl.ds(row0 + _nfull * _LANE, _tail)],
                        pooled_out.at[cid].at[pl.ds(row0 + _nfull * _LANE, _tail)])


_sc_pool = pl.kernel(
    _sc_pool_body,
    out_type=jax.ShapeDtypeStruct((_NC, _NPAD, _D), jnp.float32),
    mesh=_MESH,
    scratch_types=[
        pltpu.VMEM((_KB, _LANE), jnp.int32),
        pltpu.VMEM((_KB, _LANE), jnp.int32),
        pltpu.VMEM((_LANE, _D), jnp.float32),
        pltpu.VMEM((_LANE, _D), jnp.float32),
        pltpu.VMEM_SHARED((_NPAD, _D), jnp.float32),
        pltpu.SemaphoreType.DMA,
        pltpu.SemaphoreType.DMA,
        pltpu.SemaphoreType.DMA,
        pltpu.SemaphoreType.DMA,
    ],
)


def _sc_deg_body(dstf_hbm, deg_out, dst_f, hist_v, red_a, red_b, stage_sh):
    cid = lax.axis_index("c")
    sid = lax.axis_index("s")
    zero16 = jnp.zeros((16,), jnp.float32)
    one16 = jnp.ones((16,), jnp.float32)

    @pl.loop(0, _NPAD, step=16)
    def _(c):
        hist_v[pl.ds(c, 16)] = zero16

    ebase = (cid * _NS + sid) * _RPT * _LANE

    @pl.loop(0, _RPT // _KB)
    def _(b):
        pltpu.sync_copy(dstf_hbm.at[pl.ds(ebase + b * _FB, _FB)], dst_f)

        @pl.loop(0, _FB, step=16)
        def _(c):
            plsc.addupdate_scatter(hist_v, [dst_f[pl.ds(c, 16)]], one16)

    pltpu.sync_copy(hist_v, stage_sh.at[pl.ds(sid * _NPAD, _NPAD)])
    plsc.subcore_barrier()

    @pl.loop(0, 5)
    def _(k):
        blk = sid * 5 + k

        @pl.when(blk < _NB)
        def _():
            off = blk * _LANE
            pltpu.sync_copy(stage_sh.at[pl.ds(off, _LANE)], red_a)

            @pl.loop(1, _NS)
            def _(t):
                pltpu.sync_copy(stage_sh.at[pl.ds(t * _NPAD + off, _LANE)], red_b)

                @pl.loop(0, _LANE, step=16)
                def _(c):
                    red_a[pl.ds(c, 16)] = red_a[pl.ds(c, 16)] + red_b[pl.ds(c, 16)]

            pltpu.sync_copy(red_a, deg_out.at[pl.ds(cid * _NPAD + off, _LANE)])


_sc_deg = pl.kernel(
    _sc_deg_body,
    out_type=jax.ShapeDtypeStruct((_NC * _NPAD,), jnp.float32),
    mesh=_MESH,
    compiler_params=_SC_CP,
    scratch_types=[
        pltpu.VMEM((_FB,), jnp.int32),
        pltpu.VMEM((_NPAD,), jnp.float32),
        pltpu.VMEM((_LANE,), jnp.float32),
        pltpu.VMEM((_LANE,), jnp.float32),
        pltpu.VMEM_SHARED((_NS * _NPAD,), jnp.float32),
    ],
)


def _mlp_bn_body(p_ref, deg_ref, w1_ref, b1_ref, w2_ref, b2_ref, g_ref, bt_ref, o_ref):
    p = p_ref[0] + p_ref[1]
    deg = jnp.maximum(deg_ref[0] + deg_ref[1], 1.0)
    pooled = (p.reshape(_NB, 128, _D) / deg[:, :, None]).reshape(_NPAD, _D)
    x = jnp.dot(pooled, w1_ref[...], preferred_element_type=jnp.float32) + b1_ref[...]
    x = jnp.maximum(x, 0.0)
    x = jnp.dot(x, w2_ref[...], preferred_element_type=jnp.float32) + b2_ref[...]
    mask = lax.broadcasted_iota(jnp.int32, (_NPAD, 1), 0) < _N
    xm = jnp.where(mask, x, 0.0)
    mean = jnp.sum(xm, axis=0, keepdims=True) * (1.0 / _N)
    d = jnp.where(mask, x - mean, 0.0)
    var = jnp.sum(d * d, axis=0, keepdims=True) * (1.0 / _N)
    y = (x - mean) * lax.rsqrt(var + 1e-5) * g_ref[...] + bt_ref[...]
    o_ref[...] = jnp.maximum(y, 0.0)


_mlp_bn = pl.pallas_call(
    _mlp_bn_body,
    out_shape=jax.ShapeDtypeStruct((_NPAD, _H), jnp.float32),
)


def kernel(feats, edge_index, W1_0, b1_0, W2_0, b2_0, gamma_0, beta_0,
           W1_1, b1_1, W2_1, b2_1, gamma_1, beta_1):
    dst = edge_index[0]
    src = edge_index[1]
    npad = _EPAD - _E
    ar = jnp.arange(npad, dtype=dst.dtype)
    pad_src = ar % _N
    pad_dst = _N + (ar % (_NPAD - _N))
    src_p = jnp.concatenate([src, pad_src]).reshape(_EROWS, _LANE)
    dst_p = jnp.concatenate([dst, pad_dst]).reshape(_EROWS, _LANE)

    deg2 = _sc_deg(dst_p.reshape(-1)).reshape(_NC, _NB, _LANE)
    pooled2 = _sc_pool(feats, src_p, dst_p)
    h1 = _mlp_bn(pooled2, deg2, W1_0, b1_0.reshape(1, -1), W2_0, b2_0.reshape(1, -1),
                 gamma_0.reshape(1, -1), beta_0.reshape(1, -1))
    pooled2b = _sc_pool(h1, src_p, dst_p)
    h2 = _mlp_bn(pooled2b, deg2, W1_1, b1_1.reshape(1, -1), W2_1, b2_1.reshape(1, -1),
                 gamma_1.reshape(1, -1), beta_1.reshape(1, -1))
    return h2[:_N]
